# baseline (speedup 1.0000x reference)
import functools

import jax
import jax.numpy as jnp
import numpy as np
from jax import lax
from jax.experimental import pallas as pl
from jax.experimental.pallas import tpu as pltpu
from jax.experimental.pallas import tpu_sc as plsc

N_LEVELS = 16
F_PER_LEVEL = 2
TABLE_SIZE = 2 ** 19
_GROWTH = np.exp((np.log(4096.0) - np.log(16.0)) / (N_LEVELS - 1))
_SCALINGS = np.floor(16.0 * _GROWTH ** np.arange(N_LEVELS)).astype(np.float32)

_P1 = np.int32(2654435761 - 2 ** 32)
_P2 = np.int32(805459861)
_MASK = np.int32(TABLE_SIZE - 1)

NC = 2
NS = 16
NW = NC * NS
LANES = 16

B = 131072
NF = N_LEVELS * F_PER_LEVEL
CHUNK = 64
GROUPS = CHUNK // LANES
PER_W = B // NW
NCHUNK = PER_W // CHUNK
NIDX = CHUNK * N_LEVELS * 8


def _make_kernel():
    mesh = plsc.VectorSubcoreMesh(core_axis_name="c", subcore_axis_name="s")

    @functools.partial(
        pl.kernel,
        mesh=mesh,
        compiler_params=pltpu.CompilerParams(needs_layout_passes=False, use_tc_tiling_on_sc=False),
        out_type=jax.ShapeDtypeStruct((B * NF,), jnp.float32),
        scratch_types=[
            pltpu.VMEM((CHUNK,), jnp.float32),
            pltpu.VMEM((CHUNK,), jnp.float32),
            pltpu.VMEM((CHUNK,), jnp.float32),
            pltpu.VMEM((N_LEVELS * CHUNK,), jnp.float32),
            pltpu.VMEM((N_LEVELS * CHUNK,), jnp.float32),
            pltpu.VMEM((N_LEVELS * CHUNK,), jnp.float32),
            pltpu.VMEM((NIDX,), jnp.int32),
            pltpu.VMEM((NIDX, F_PER_LEVEL), jnp.float32),
            pltpu.VMEM((CHUNK * NF,), jnp.float32),
            pltpu.SemaphoreType.DMA,
        ],
    )
    def sc_kernel(xs_hbm, ys_hbm, zs_hbm, table_hbm, out_hbm,
                  xv, yv, zv, oxv, oyv, ozv, idxv,
                  rowsv, outv, sem0):
        wid = lax.axis_index("s") * NC + lax.axis_index("c")
        iota = lax.iota(jnp.int32, LANES)
        zeros16 = jnp.zeros((LANES,), jnp.int32)
        ones16 = zeros16 + 1

        def chunk_body(c, carry):
            pbase = wid * PER_W + c * CHUNK
            pltpu.sync_copy(xs_hbm.at[pl.ds(pbase, CHUNK)], xv)
            pltpu.sync_copy(ys_hbm.at[pl.ds(pbase, CHUNK)], yv)
            pltpu.sync_copy(zs_hbm.at[pl.ds(pbase, CHUNK)], zv)

            def hash_group(g, carry):
                s = g * LANES
                x16 = xv[pl.ds(s, LANES)]
                y16 = yv[pl.ds(s, LANES)]
                z16 = zv[pl.ds(s, LANES)]
                for l in range(N_LEVELS):
                    sl = float(_SCALINGS[l])
                    sx = x16 * sl
                    sy = y16 * sl
                    sz = z16 * sl
                    xf = sx.astype(jnp.int32)
                    yf = sy.astype(jnp.int32)
                    zf = sz.astype(jnp.int32)
                    xff = xf.astype(jnp.float32)
                    yff = yf.astype(jnp.float32)
                    zff = zf.astype(jnp.float32)
                    oxv[pl.ds(l * CHUNK + s, LANES)] = sx - xff
                    oyv[pl.ds(l * CHUNK + s, LANES)] = sy - yff
                    ozv[pl.ds(l * CHUNK + s, LANES)] = sz - zff
                    xc = jnp.where(sx > xff, xf + 1, xf)
                    yc = jnp.where(sy > yff, yf + 1, yf)
                    zc = jnp.where(sz > zff, zf + 1, zf)
                    hyf = yf * _P1
                    hyc = yc * _P1
                    hzf = zf * _P2
                    hzc = zc * _P2
                    ycc = hyc ^ hzc
                    yfc = hyf ^ hzc
                    yff_ = hyf ^ hzf
                    ycf = hyc ^ hzf
                    lofs = np.int32(l * TABLE_SIZE)
                    rb = g * (N_LEVELS * 128) + l * 128
                    combos = (xc ^ ycc, xc ^ yfc, xf ^ yfc, xf ^ ycc,
                              xc ^ ycf, xc ^ yff_, xf ^ yff_, xf ^ ycf)
                    for corner, hv in enumerate(combos):
                        idxv[pl.ds(rb + corner * LANES, LANES)] = (
                            (hv & _MASK) + lofs)
                return carry

            lax.fori_loop(np.int32(0), np.int32(GROUPS), hash_group,
                          np.int32(0))

            pltpu.async_copy(table_hbm.at[idxv], rowsv, sem0).wait()

            def interp_group(g, carry):
                s = g * LANES
                p32 = (iota + s) * NF
                for l in range(N_LEVELS):
                    ox = oxv[pl.ds(l * CHUNK + s, LANES)]
                    oy = oyv[pl.ds(l * CHUNK + s, LANES)]
                    oz = ozv[pl.ds(l * CHUNK + s, LANES)]
                    mx = 1.0 - ox
                    my = 1.0 - oy
                    mz = 1.0 - oz
                    rb = g * (N_LEVELS * 128) + l * 128
                    for ft, fsel in ((0, zeros16), (1, ones16)):
                        f = [plsc.load_gather(
                                 rowsv, [iota + (rb + corner * LANES), fsel])
                             for corner in range(8)]
                        f03 = f[0] * ox + f[3] * mx
                        f12 = f[1] * ox + f[2] * mx
                        f56 = f[5] * ox + f[6] * mx
                        f47 = f[4] * ox + f[7] * mx
                        f0312 = f03 * oy + f12 * my
                        f4756 = f47 * oy + f56 * my
                        enc = f0312 * oz + f4756 * mz
                        plsc.store_scatter(outv, [p32 + (2 * l + ft)], enc)
                return carry

            lax.fori_loop(np.int32(0), np.int32(GROUPS), interp_group,
                          np.int32(0))

            pltpu.sync_copy(outv, out_hbm.at[pl.ds(pbase * NF, CHUNK * NF)])
            return carry

        lax.fori_loop(np.int32(0), np.int32(NCHUNK), chunk_body, np.int32(0))

    return sc_kernel


_sc_kernel_cache = []


@jax.jit
def _run(in_tensor, hash_table):
    if not _sc_kernel_cache:
        _sc_kernel_cache.append(_make_kernel())
    coords = in_tensor.T
    flat = _sc_kernel_cache[0](coords[0], coords[1], coords[2], hash_table)
    return flat.reshape(B, NF)


def kernel(in_tensor, hash_table):
    with jax.enable_x64(False):
        return _run(in_tensor, hash_table)


# speedup vs baseline: 10.1360x; 10.1360x over previous
import functools

import jax
import jax.numpy as jnp
import numpy as np
from jax import lax
from jax.experimental import pallas as pl
from jax.experimental.pallas import tpu as pltpu
from jax.experimental.pallas import tpu_sc as plsc

N_LEVELS = 16
F_PER_LEVEL = 2
TABLE_SIZE = 2 ** 19
_GROWTH = np.exp((np.log(4096.0) - np.log(16.0)) / (N_LEVELS - 1))
_SCALINGS = np.floor(16.0 * _GROWTH ** np.arange(N_LEVELS)).astype(np.float32)

_P1 = np.int32(2654435761 - 2 ** 32)
_P2 = np.int32(805459861)
_MASK = np.int32(TABLE_SIZE - 1)
_BLK = np.int32(-128)

NC = 2
NS = 16
NW = NC * NS
LANES = 16

B = 131072
NF = N_LEVELS * F_PER_LEVEL
CHUNK = 128
GROUPS = CHUNK // LANES
PER_W = B // NW
NCHUNK = PER_W // CHUNK
NIDX = CHUNK * N_LEVELS * 8


def _make_kernel():
    mesh = plsc.VectorSubcoreMesh(core_axis_name="c", subcore_axis_name="s")

    @functools.partial(
        pl.kernel,
        mesh=mesh,
        compiler_params=pltpu.CompilerParams(needs_layout_passes=False),
        out_type=jax.ShapeDtypeStruct((B * NF,), jnp.float32),
        scratch_types=[
            pltpu.VMEM((CHUNK,), jnp.float32),
            pltpu.VMEM((CHUNK,), jnp.float32),
            pltpu.VMEM((CHUNK,), jnp.float32),
            pltpu.VMEM((N_LEVELS * CHUNK,), jnp.float32),
            pltpu.VMEM((N_LEVELS * CHUNK,), jnp.float32),
            pltpu.VMEM((N_LEVELS * CHUNK,), jnp.float32),
            pltpu.VMEM((NIDX,), jnp.int32),
            pltpu.VMEM((NIDX,), jnp.int32),
            pltpu.VMEM((NIDX,), jnp.float32),
            pltpu.VMEM((NIDX,), jnp.float32),
            pltpu.VMEM((CHUNK * NF,), jnp.float32),
            pltpu.SemaphoreType.DMA,
            pltpu.SemaphoreType.DMA,
        ],
    )
    def sc_kernel(xs_hbm, ys_hbm, zs_hbm, table_hbm, out_hbm,
                  xv, yv, zv, oxv, oyv, ozv, idx0v, idx1v,
                  rows0v, rows1v, outv, sem0, sem1):
        wid = lax.axis_index("s") * NC + lax.axis_index("c")
        iota = lax.iota(jnp.int32, LANES)

        def chunk_body(c, carry):
            pbase = wid * PER_W + c * CHUNK
            pltpu.sync_copy(xs_hbm.at[pl.ds(pbase, CHUNK)], xv)
            pltpu.sync_copy(ys_hbm.at[pl.ds(pbase, CHUNK)], yv)
            pltpu.sync_copy(zs_hbm.at[pl.ds(pbase, CHUNK)], zv)

            def hash_group(g, carry):
                s = g * LANES
                x16 = xv[pl.ds(s, LANES)]
                y16 = yv[pl.ds(s, LANES)]
                z16 = zv[pl.ds(s, LANES)]
                for l in range(N_LEVELS):
                    sl = float(_SCALINGS[l])
                    sx = x16 * sl
                    sy = y16 * sl
                    sz = z16 * sl
                    xf = sx.astype(jnp.int32)
                    yf = sy.astype(jnp.int32)
                    zf = sz.astype(jnp.int32)
                    xff = xf.astype(jnp.float32)
                    yff = yf.astype(jnp.float32)
                    zff = zf.astype(jnp.float32)
                    oxv[pl.ds(l * CHUNK + s, LANES)] = sx - xff
                    oyv[pl.ds(l * CHUNK + s, LANES)] = sy - yff
                    ozv[pl.ds(l * CHUNK + s, LANES)] = sz - zff
                    xc = jnp.where(sx > xff, xf + 1, xf)
                    yc = jnp.where(sy > yff, yf + 1, yf)
                    zc = jnp.where(sz > zff, zf + 1, zf)
                    hyf = yf * _P1
                    hyc = yc * _P1
                    hzf = zf * _P2
                    hzc = zc * _P2
                    ycc = hyc ^ hzc
                    yfc = hyf ^ hzc
                    yff_ = hyf ^ hzf
                    ycf = hyc ^ hzf
                    lofs = np.int32(l * TABLE_SIZE)
                    rb = g * (N_LEVELS * 128) + l * 128
                    combos = (xc ^ ycc, xc ^ yfc, xf ^ yfc, xf ^ ycc,
                              xc ^ ycf, xc ^ yff_, xf ^ yff_, xf ^ ycf)
                    for corner, hv in enumerate(combos):
                        h = (hv & _MASK) + lofs
                        e0 = h + (h & _BLK)
                        idx0v[pl.ds(rb + corner * LANES, LANES)] = e0
                        idx1v[pl.ds(rb + corner * LANES, LANES)] = e0 + 128
                return carry

            lax.fori_loop(np.int32(0), np.int32(GROUPS), hash_group,
                          np.int32(0))

            cp0 = pltpu.async_copy(table_hbm.at[idx0v], rows0v, sem0)
            cp1 = pltpu.async_copy(table_hbm.at[idx1v], rows1v, sem1)
            cp0.wait()
            cp1.wait()

            def interp_group(g, carry):
                s = g * LANES
                p32 = (iota + s) * NF
                for l in range(N_LEVELS):
                    ox = oxv[pl.ds(l * CHUNK + s, LANES)]
                    oy = oyv[pl.ds(l * CHUNK + s, LANES)]
                    oz = ozv[pl.ds(l * CHUNK + s, LANES)]
                    mx = 1.0 - ox
                    my = 1.0 - oy
                    mz = 1.0 - oz
                    rb = g * (N_LEVELS * 128) + l * 128
                    for ft, rv in ((0, rows0v), (1, rows1v)):
                        f = [rv[pl.ds(rb + corner * LANES, LANES)]
                             for corner in range(8)]
                        f03 = f[0] * ox + f[3] * mx
                        f12 = f[1] * ox + f[2] * mx
                        f56 = f[5] * ox + f[6] * mx
                        f47 = f[4] * ox + f[7] * mx
                        f0312 = f03 * oy + f12 * my
                        f4756 = f47 * oy + f56 * my
                        enc = f0312 * oz + f4756 * mz
                        plsc.store_scatter(outv, [p32 + (2 * l + ft)], enc)
                return carry

            lax.fori_loop(np.int32(0), np.int32(GROUPS), interp_group,
                          np.int32(0))

            pltpu.sync_copy(outv, out_hbm.at[pl.ds(pbase * NF, CHUNK * NF)])
            return carry

        lax.fori_loop(np.int32(0), np.int32(NCHUNK), chunk_body, np.int32(0))

    return sc_kernel


_sc_kernel_cache = []


@jax.jit
def _run(in_tensor, hash_table):
    if not _sc_kernel_cache:
        _sc_kernel_cache.append(_make_kernel())
    coords = in_tensor.T
    tflat = hash_table.reshape(65536, 128, 2).transpose(0, 2, 1).reshape(-1)
    flat = _sc_kernel_cache[0](coords[0], coords[1], coords[2], tflat)
    return flat.reshape(B, NF)


def kernel(in_tensor, hash_table):
    with jax.enable_x64(False):
        return _run(in_tensor, hash_table)


# speedup vs baseline: 10.7638x; 1.0619x over previous
import functools

import jax
import jax.numpy as jnp
import numpy as np
from jax import lax
from jax.experimental import pallas as pl
from jax.experimental.pallas import tpu as pltpu
from jax.experimental.pallas import tpu_sc as plsc

N_LEVELS = 16
F_PER_LEVEL = 2
TABLE_SIZE = 2 ** 19
_GROWTH = np.exp((np.log(4096.0) - np.log(16.0)) / (N_LEVELS - 1))
_SCALINGS = np.floor(16.0 * _GROWTH ** np.arange(N_LEVELS)).astype(np.float32)

_P1 = np.int32(2654435761 - 2 ** 32)
_P2 = np.int32(805459861)
_MASK = np.int32(TABLE_SIZE - 1)
_BLK = np.int32(-128)

NC = 2
NS = 16
NW = NC * NS
LANES = 16

B = 131072
NF = N_LEVELS * F_PER_LEVEL
CHUNK = 64
GROUPS = CHUNK // LANES
PER_W = B // NW
NCHUNK = PER_W // CHUNK
NIDX = CHUNK * N_LEVELS * 8
ROWSTRIDE = N_LEVELS * 128


def _make_kernel():
    mesh = plsc.VectorSubcoreMesh(core_axis_name="c", subcore_axis_name="s")

    @functools.partial(
        pl.kernel,
        mesh=mesh,
        compiler_params=pltpu.CompilerParams(needs_layout_passes=False),
        out_type=jax.ShapeDtypeStruct((B * NF,), jnp.float32),
        scratch_types=[
            pltpu.VMEM((PER_W,), jnp.float32),
            pltpu.VMEM((PER_W,), jnp.float32),
            pltpu.VMEM((PER_W,), jnp.float32),
            pltpu.VMEM((N_LEVELS * CHUNK,), jnp.float32),
            pltpu.VMEM((N_LEVELS * CHUNK,), jnp.float32),
            pltpu.VMEM((N_LEVELS * CHUNK,), jnp.float32),
            pltpu.VMEM((N_LEVELS * CHUNK,), jnp.float32),
            pltpu.VMEM((N_LEVELS * CHUNK,), jnp.float32),
            pltpu.VMEM((N_LEVELS * CHUNK,), jnp.float32),
            pltpu.VMEM((NIDX,), jnp.int32),
            pltpu.VMEM((NIDX,), jnp.int32),
            pltpu.VMEM((NIDX,), jnp.int32),
            pltpu.VMEM((NIDX,), jnp.int32),
            pltpu.VMEM((NIDX,), jnp.float32),
            pltpu.VMEM((NIDX,), jnp.float32),
            pltpu.VMEM((NIDX,), jnp.float32),
            pltpu.VMEM((NIDX,), jnp.float32),
            pltpu.VMEM((CHUNK * NF,), jnp.float32),
            pltpu.VMEM((CHUNK * NF,), jnp.float32),
            pltpu.SemaphoreType.DMA,
            pltpu.SemaphoreType.DMA,
            pltpu.SemaphoreType.DMA,
            pltpu.SemaphoreType.DMA,
        ],
    )
    def sc_kernel(xs_hbm, ys_hbm, zs_hbm, table_hbm, out_hbm,
                  xv, yv, zv,
                  oxA, oyA, ozA, oxB, oyB, ozB,
                  idx0A, idx1A, idx0B, idx1B,
                  rows0A, rows1A, rows0B, rows1B,
                  outA, outB, sem0A, sem1A, sem0B, sem1B):
        wid = lax.axis_index("s") * NC + lax.axis_index("c")
        iota = lax.iota(jnp.int32, LANES)
        wbase = wid * PER_W

        pltpu.sync_copy(xs_hbm.at[pl.ds(wbase, PER_W)], xv)
        pltpu.sync_copy(ys_hbm.at[pl.ds(wbase, PER_W)], yv)
        pltpu.sync_copy(zs_hbm.at[pl.ds(wbase, PER_W)], zv)

        bufs = (
            (oxA, oyA, ozA, idx0A, idx1A, rows0A, rows1A, outA, sem0A, sem1A),
            (oxB, oyB, ozB, idx0B, idx1B, rows0B, rows1B, outB, sem0B, sem1B),
        )

        def pass1(c, buf):
            ox_, oy_, oz_, i0, i1 = buf[0], buf[1], buf[2], buf[3], buf[4]
            cbase = c * CHUNK

            def hash_group(g, carry):
                s = g * LANES
                x16 = xv[pl.ds(cbase + s, LANES)]
                y16 = yv[pl.ds(cbase + s, LANES)]
                z16 = zv[pl.ds(cbase + s, LANES)]
                for l in range(N_LEVELS):
                    sl = float(_SCALINGS[l])
                    sx = x16 * sl
                    sy = y16 * sl
                    sz = z16 * sl
                    xf = sx.astype(jnp.int32)
                    yf = sy.astype(jnp.int32)
                    zf = sz.astype(jnp.int32)
                    xff = xf.astype(jnp.float32)
                    yff = yf.astype(jnp.float32)
                    zff = zf.astype(jnp.float32)
                    ox_[pl.ds(l * CHUNK + s, LANES)] = sx - xff
                    oy_[pl.ds(l * CHUNK + s, LANES)] = sy - yff
                    oz_[pl.ds(l * CHUNK + s, LANES)] = sz - zff
                    xc = jnp.where(sx > xff, xf + 1, xf)
                    yc = jnp.where(sy > yff, yf + 1, yf)
                    zc = jnp.where(sz > zff, zf + 1, zf)
                    hyf = yf * _P1
                    hyc = yc * _P1
                    hzf = zf * _P2
                    hzc = zc * _P2
                    ycc = hyc ^ hzc
                    yfc = hyf ^ hzc
                    yff_ = hyf ^ hzf
                    ycf = hyc ^ hzf
                    lofs = np.int32(l * TABLE_SIZE)
                    rb = g * ROWSTRIDE + l * 128
                    combos = (xc ^ ycc, xc ^ yfc, xf ^ yfc, xf ^ ycc,
                              xc ^ ycf, xc ^ yff_, xf ^ yff_, xf ^ ycf)
                    for corner, hv in enumerate(combos):
                        h = (hv & _MASK) + lofs
                        e0 = h + (h & _BLK)
                        i0[pl.ds(rb + corner * LANES, LANES)] = e0
                        i1[pl.ds(rb + corner * LANES, LANES)] = e0 + 128
                return carry

            lax.fori_loop(np.int32(0), np.int32(GROUPS), hash_group,
                          np.int32(0))

        def start_gather(buf):
            i0, i1, r0, r1, s0, s1 = buf[3], buf[4], buf[5], buf[6], buf[8], buf[9]
            pltpu.async_copy(table_hbm.at[i0], r0, s0)
            pltpu.async_copy(table_hbm.at[i1], r1, s1)

        def wait_gather(buf):
            i0, i1, r0, r1, s0, s1 = buf[3], buf[4], buf[5], buf[6], buf[8], buf[9]
            pltpu.make_async_copy(table_hbm.at[i0], r0, s0).wait()
            pltpu.make_async_copy(table_hbm.at[i1], r1, s1).wait()

        def pass2(c, buf):
            ox_, oy_, oz_, r0, r1, outv = (buf[0], buf[1], buf[2],
                                           buf[5], buf[6], buf[7])

            def interp_group(g, carry):
                s = g * LANES
                p32 = (iota + s) * NF
                for l in range(N_LEVELS):
                    ox = ox_[pl.ds(l * CHUNK + s, LANES)]
                    oy = oy_[pl.ds(l * CHUNK + s, LANES)]
                    oz = oz_[pl.ds(l * CHUNK + s, LANES)]
                    mx = 1.0 - ox
                    my = 1.0 - oy
                    mz = 1.0 - oz
                    rb = g * ROWSTRIDE + l * 128
                    for ft, rv in ((0, r0), (1, r1)):
                        f = [rv[pl.ds(rb + corner * LANES, LANES)]
                             for corner in range(8)]
                        f03 = f[0] * ox + f[3] * mx
                        f12 = f[1] * ox + f[2] * mx
                        f56 = f[5] * ox + f[6] * mx
                        f47 = f[4] * ox + f[7] * mx
                        f0312 = f03 * oy + f12 * my
                        f4756 = f47 * oy + f56 * my
                        enc = f0312 * oz + f4756 * mz
                        plsc.store_scatter(outv, [p32 + (2 * l + ft)], enc)
                return carry

            lax.fori_loop(np.int32(0), np.int32(GROUPS), interp_group,
                          np.int32(0))
            pltpu.sync_copy(
                outv, out_hbm.at[pl.ds((wbase + c * CHUNK) * NF, CHUNK * NF)])

        A, Bb = bufs

        pass1(np.int32(0), A)
        start_gather(A)

        def pair_body(c2, carry):
            cA = c2 * np.int32(2)
            pass1(cA + 1, Bb)
            start_gather(Bb)
            wait_gather(A)
            pass2(cA, A)
            pass1(cA + 2, A)
            start_gather(A)
            wait_gather(Bb)
            pass2(cA + 1, Bb)
            return carry

        lax.fori_loop(np.int32(0), np.int32(NCHUNK // 2 - 1), pair_body,
                      np.int32(0))

        last = np.int32(NCHUNK - 2)
        pass1(last + 1, Bb)
        start_gather(Bb)
        wait_gather(A)
        pass2(last, A)
        wait_gather(Bb)
        pass2(last + 1, Bb)

    return sc_kernel


_sc_kernel_cache = []


@jax.jit
def _run(in_tensor, hash_table):
    if not _sc_kernel_cache:
        _sc_kernel_cache.append(_make_kernel())
    coords = in_tensor.T
    tflat = hash_table.reshape(65536, 128, 2).transpose(0, 2, 1).reshape(-1)
    flat = _sc_kernel_cache[0](coords[0], coords[1], coords[2], tflat)
    return flat.reshape(B, NF)


def kernel(in_tensor, hash_table):
    with jax.enable_x64(False):
        return _run(in_tensor, hash_table)


# speedup vs baseline: 12.8656x; 1.1953x over previous
import functools

import jax
import jax.numpy as jnp
import numpy as np
from jax import lax
from jax.experimental import pallas as pl
from jax.experimental.pallas import tpu as pltpu
from jax.experimental.pallas import tpu_sc as plsc

N_LEVELS = 16
F_PER_LEVEL = 2
TABLE_SIZE = 2 ** 19
_GROWTH = np.exp((np.log(4096.0) - np.log(16.0)) / (N_LEVELS - 1))
_SCALINGS = np.floor(16.0 * _GROWTH ** np.arange(N_LEVELS)).astype(np.float32)

_P1 = np.int32(2654435761 - 2 ** 32)
_P2 = np.int32(805459861)
_MASK = np.int32(TABLE_SIZE - 1)
_BLK = np.int32(-128)

NC = 2
NS = 16
NW = NC * NS
LANES = 16

B = 131072
NF = N_LEVELS * F_PER_LEVEL
CHUNK = 128
GROUPS = CHUNK // LANES
PER_W = B // NW
NCHUNK = PER_W // CHUNK
NIDX = CHUNK * N_LEVELS * 8
ROWSTRIDE = N_LEVELS * 128


def _make_kernel():
    mesh = plsc.VectorSubcoreMesh(core_axis_name="c", subcore_axis_name="s")

    @functools.partial(
        pl.kernel,
        mesh=mesh,
        compiler_params=pltpu.CompilerParams(needs_layout_passes=False),
        out_type=jax.ShapeDtypeStruct((B * NF,), jnp.float32),
        scratch_types=[
            pltpu.VMEM((PER_W,), jnp.float32),
            pltpu.VMEM((PER_W,), jnp.float32),
            pltpu.VMEM((PER_W,), jnp.float32),
            pltpu.VMEM((N_LEVELS * CHUNK,), jnp.float32),
            pltpu.VMEM((N_LEVELS * CHUNK,), jnp.float32),
            pltpu.VMEM((N_LEVELS * CHUNK,), jnp.float32),
            pltpu.VMEM((N_LEVELS * CHUNK,), jnp.float32),
            pltpu.VMEM((N_LEVELS * CHUNK,), jnp.float32),
            pltpu.VMEM((N_LEVELS * CHUNK,), jnp.float32),
            pltpu.VMEM((NIDX,), jnp.int32),
            pltpu.VMEM((NIDX,), jnp.int32),
            pltpu.VMEM((NIDX,), jnp.int32),
            pltpu.VMEM((NIDX,), jnp.int32),
            pltpu.VMEM((CHUNK * NF,), jnp.float32),
            pltpu.VMEM((CHUNK * NF,), jnp.float32),
            pltpu.SemaphoreType.DMA,
            pltpu.SemaphoreType.DMA,
        ],
    )
    def sc_kernel(xs_hbm, ys_hbm, zs_hbm, table_hbm, out_hbm,
                  xv, yv, zv,
                  oxA, oyA, ozA, oxB, oyB, ozB,
                  idxA, idxB, rowsA, rowsB,
                  outA, outB, semA, semB):
        wid = lax.axis_index("s") * NC + lax.axis_index("c")
        iota = lax.iota(jnp.int32, LANES)
        wbase = wid * PER_W

        pltpu.sync_copy(xs_hbm.at[pl.ds(wbase, PER_W)], xv)
        pltpu.sync_copy(ys_hbm.at[pl.ds(wbase, PER_W)], yv)
        pltpu.sync_copy(zs_hbm.at[pl.ds(wbase, PER_W)], zv)

        bufs = (
            (oxA, oyA, ozA, idxA, rowsA, outA, semA),
            (oxB, oyB, ozB, idxB, rowsB, outB, semB),
        )

        def pass1(c, buf):
            ox_, oy_, oz_, i0 = buf[0], buf[1], buf[2], buf[3]
            cbase = c * CHUNK

            def hash_group(g, carry):
                s = g * LANES
                x16 = xv[pl.ds(cbase + s, LANES)]
                y16 = yv[pl.ds(cbase + s, LANES)]
                z16 = zv[pl.ds(cbase + s, LANES)]
                for l in range(N_LEVELS):
                    sl = float(_SCALINGS[l])
                    sx = x16 * sl
                    sy = y16 * sl
                    sz = z16 * sl
                    xf = sx.astype(jnp.int32)
                    yf = sy.astype(jnp.int32)
                    zf = sz.astype(jnp.int32)
                    xff = xf.astype(jnp.float32)
                    yff = yf.astype(jnp.float32)
                    zff = zf.astype(jnp.float32)
                    ox_[pl.ds(l * CHUNK + s, LANES)] = sx - xff
                    oy_[pl.ds(l * CHUNK + s, LANES)] = sy - yff
                    oz_[pl.ds(l * CHUNK + s, LANES)] = sz - zff
                    xc = jnp.where(sx > xff, xf + 1, xf)
                    yc = jnp.where(sy > yff, yf + 1, yf)
                    zc = jnp.where(sz > zff, zf + 1, zf)
                    hyf = yf * _P1
                    hyc = yc * _P1
                    hzf = zf * _P2
                    hzc = zc * _P2
                    ycc = hyc ^ hzc
                    yfc = hyf ^ hzc
                    yff_ = hyf ^ hzf
                    ycf = hyc ^ hzf
                    lofs = np.int32(l * TABLE_SIZE)
                    rb = g * ROWSTRIDE + l * 128
                    combos = (xc ^ ycc, xc ^ yfc, xf ^ yfc, xf ^ ycc,
                              xc ^ ycf, xc ^ yff_, xf ^ yff_, xf ^ ycf)
                    for corner, hv in enumerate(combos):
                        i0[pl.ds(rb + corner * LANES, LANES)] = (
                            (hv & _MASK) + lofs)
                return carry

            lax.fori_loop(np.int32(0), np.int32(GROUPS), hash_group,
                          np.int32(0))

        def start_gather(buf):
            i0, rv, sem = buf[3], buf[4], buf[6]
            pltpu.async_copy(table_hbm.at[i0], rv, sem)

        def wait_gather(buf):
            i0, rv, sem = buf[3], buf[4], buf[6]
            pltpu.make_async_copy(table_hbm.at[i0], rv, sem).wait()

        def pass2(c, buf):
            ox_, oy_, oz_, rv, outv = (buf[0], buf[1], buf[2],
                                       buf[4], buf[5])

            def interp_group(g, carry):
                s = g * LANES
                p32 = (iota + s) * NF
                for l in range(N_LEVELS):
                    ox = ox_[pl.ds(l * CHUNK + s, LANES)]
                    oy = oy_[pl.ds(l * CHUNK + s, LANES)]
                    oz = oz_[pl.ds(l * CHUNK + s, LANES)]
                    mx = 1.0 - ox
                    my = 1.0 - oy
                    mz = 1.0 - oz
                    rb = g * ROWSTRIDE + l * 128
                    f = []
                    for corner in range(8):
                        packed = rv[pl.ds(rb + corner * LANES, LANES)]
                        bf = plsc.bitcast(packed, jnp.bfloat16)
                        f.append(plsc.unpack(
                            bf, format=plsc.PackFormat.INTERLEAVED))
                    for ft in range(F_PER_LEVEL):
                        f03 = f[0][ft] * ox + f[3][ft] * mx
                        f12 = f[1][ft] * ox + f[2][ft] * mx
                        f56 = f[5][ft] * ox + f[6][ft] * mx
                        f47 = f[4][ft] * ox + f[7][ft] * mx
                        f0312 = f03 * oy + f12 * my
                        f4756 = f47 * oy + f56 * my
                        enc = f0312 * oz + f4756 * mz
                        plsc.store_scatter(outv, [p32 + (2 * l + ft)], enc)
                return carry

            lax.fori_loop(np.int32(0), np.int32(GROUPS), interp_group,
                          np.int32(0))
            pltpu.sync_copy(
                outv, out_hbm.at[pl.ds((wbase + c * CHUNK) * NF, CHUNK * NF)])

        A, Bb = bufs

        pass1(np.int32(0), A)
        start_gather(A)

        def pair_body(c2, carry):
            cA = c2 * np.int32(2)
            pass1(cA + 1, Bb)
            start_gather(Bb)
            wait_gather(A)
            pass2(cA, A)
            pass1(cA + 2, A)
            start_gather(A)
            wait_gather(Bb)
            pass2(cA + 1, Bb)
            return carry

        lax.fori_loop(np.int32(0), np.int32(NCHUNK // 2 - 1), pair_body,
                      np.int32(0))

        last = np.int32(NCHUNK - 2)
        pass1(last + 1, Bb)
        start_gather(Bb)
        wait_gather(A)
        pass2(last, A)
        wait_gather(Bb)
        pass2(last + 1, Bb)

    return sc_kernel


_sc_kernel_cache = []


@jax.jit
def _run(in_tensor, hash_table):
    if not _sc_kernel_cache:
        _sc_kernel_cache.append(_make_kernel())
    coords = in_tensor.T
    tpacked = lax.bitcast_convert_type(
        hash_table.astype(jnp.bfloat16), jnp.int32)
    flat = _sc_kernel_cache[0](coords[0], coords[1], coords[2], tpacked)
    return flat.reshape(B, NF)


def kernel(in_tensor, hash_table):
    with jax.enable_x64(False):
        return _run(in_tensor, hash_table)


# speedup vs baseline: 16.9791x; 1.3197x over previous
import functools

import jax
import jax.numpy as jnp
import numpy as np
from jax import lax
from jax.experimental import pallas as pl
from jax.experimental.pallas import tpu as pltpu
from jax.experimental.pallas import tpu_sc as plsc

N_LEVELS = 16
F_PER_LEVEL = 2
TABLE_SIZE = 2 ** 19
_GROWTH = np.exp((np.log(4096.0) - np.log(16.0)) / (N_LEVELS - 1))
_SCALINGS = np.floor(16.0 * _GROWTH ** np.arange(N_LEVELS)).astype(np.float32)

_P1 = np.int32(2654435761 - 2 ** 32)
_P2 = np.int32(805459861)
_MASK = np.int32(TABLE_SIZE - 1)
_BLK = np.int32(-128)

NC = 2
NS = 16
NW = NC * NS
LANES = 16

B = 131072
NF = N_LEVELS * F_PER_LEVEL
CHUNK = 128
GROUPS = CHUNK // LANES
PER_W = B // NW
NCHUNK = PER_W // CHUNK
NIDX = CHUNK * N_LEVELS * 8
ROWSTRIDE = N_LEVELS * 128


def _make_kernel():
    mesh = plsc.VectorSubcoreMesh(core_axis_name="c", subcore_axis_name="s")

    @functools.partial(
        pl.kernel,
        mesh=mesh,
        compiler_params=pltpu.CompilerParams(needs_layout_passes=False),
        out_type=jax.ShapeDtypeStruct((B * NF,), jnp.float32),
        scratch_types=[
            pltpu.VMEM((PER_W,), jnp.float32),
            pltpu.VMEM((PER_W,), jnp.float32),
            pltpu.VMEM((PER_W,), jnp.float32),
            pltpu.VMEM((N_LEVELS * CHUNK,), jnp.float32),
            pltpu.VMEM((N_LEVELS * CHUNK,), jnp.float32),
            pltpu.VMEM((N_LEVELS * CHUNK,), jnp.float32),
            pltpu.VMEM((N_LEVELS * CHUNK,), jnp.float32),
            pltpu.VMEM((N_LEVELS * CHUNK,), jnp.float32),
            pltpu.VMEM((N_LEVELS * CHUNK,), jnp.float32),
            pltpu.VMEM((NIDX,), jnp.int32),
            pltpu.VMEM((NIDX,), jnp.int32),
            pltpu.VMEM((NIDX,), jnp.int32),
            pltpu.VMEM((NIDX,), jnp.int32),
            pltpu.VMEM((CHUNK * NF,), jnp.float32),
            pltpu.VMEM((CHUNK * NF,), jnp.float32),
            pltpu.SemaphoreType.DMA,
            pltpu.SemaphoreType.DMA,
        ],
    )
    def sc_kernel(xs_hbm, ys_hbm, zs_hbm, table_hbm, out_hbm,
                  xv, yv, zv,
                  oxA, oyA, ozA, oxB, oyB, ozB,
                  idxA, idxB, rowsA, rowsB,
                  outA, outB, semA, semB):
        wid = lax.axis_index("s") * NC + lax.axis_index("c")
        iota = lax.iota(jnp.int32, LANES)
        wbase = wid * PER_W

        pltpu.sync_copy(xs_hbm.at[pl.ds(wbase, PER_W)], xv)
        pltpu.sync_copy(ys_hbm.at[pl.ds(wbase, PER_W)], yv)
        pltpu.sync_copy(zs_hbm.at[pl.ds(wbase, PER_W)], zv)

        bufs = (
            (oxA, oyA, ozA, idxA, rowsA, outA, semA),
            (oxB, oyB, ozB, idxB, rowsB, outB, semB),
        )

        def pass1(c, buf):
            ox_, oy_, oz_, i0 = buf[0], buf[1], buf[2], buf[3]
            cbase = c * CHUNK

            def hash_group(g, carry):
                s = g * LANES
                x16 = xv[pl.ds(cbase + s, LANES)]
                y16 = yv[pl.ds(cbase + s, LANES)]
                z16 = zv[pl.ds(cbase + s, LANES)]
                for l in range(N_LEVELS):
                    sl = float(_SCALINGS[l])
                    sx = x16 * sl
                    sy = y16 * sl
                    sz = z16 * sl
                    xf = sx.astype(jnp.int32)
                    yf = sy.astype(jnp.int32)
                    zf = sz.astype(jnp.int32)
                    xff = xf.astype(jnp.float32)
                    yff = yf.astype(jnp.float32)
                    zff = zf.astype(jnp.float32)
                    ox_[pl.ds(l * CHUNK + s, LANES)] = sx - xff
                    oy_[pl.ds(l * CHUNK + s, LANES)] = sy - yff
                    oz_[pl.ds(l * CHUNK + s, LANES)] = sz - zff
                    xc = jnp.where(sx > xff, xf + 1, xf)
                    yc = jnp.where(sy > yff, yf + 1, yf)
                    zc = jnp.where(sz > zff, zf + 1, zf)
                    hyf = yf * _P1
                    hyc = yc * _P1
                    hzf = zf * _P2
                    hzc = zc * _P2
                    ycc = hyc ^ hzc
                    yfc = hyf ^ hzc
                    yff_ = hyf ^ hzf
                    ycf = hyc ^ hzf
                    lofs = np.int32(l * TABLE_SIZE)
                    rb = g * ROWSTRIDE + l * 128
                    combos = (xc ^ ycc, xc ^ yfc, xf ^ yfc, xf ^ ycc,
                              xc ^ ycf, xc ^ yff_, xf ^ yff_, xf ^ ycf)
                    for corner, hv in enumerate(combos):
                        i0[pl.ds(rb + corner * LANES, LANES)] = (
                            (hv & _MASK) + lofs)
                return carry

            lax.fori_loop(np.int32(0), np.int32(GROUPS), hash_group,
                          np.int32(0))

        def start_gather(buf):
            i0, rv, sem = buf[3], buf[4], buf[6]
            pltpu.async_copy(table_hbm.at[i0], rv, sem)

        def wait_gather(buf):
            i0, rv, sem = buf[3], buf[4], buf[6]
            pltpu.make_async_copy(table_hbm.at[i0], rv, sem).wait()

        def pass2(c, buf):
            ox_, oy_, oz_, rv, outv = (buf[0], buf[1], buf[2],
                                       buf[4], buf[5])

            def interp_group(g, carry):
                s = g * LANES
                p32 = (iota + s) * NF
                for l in range(N_LEVELS):
                    ox = ox_[pl.ds(l * CHUNK + s, LANES)]
                    oy = oy_[pl.ds(l * CHUNK + s, LANES)]
                    oz = oz_[pl.ds(l * CHUNK + s, LANES)]
                    mx = 1.0 - ox
                    my = 1.0 - oy
                    mz = 1.0 - oz
                    rb = g * ROWSTRIDE + l * 128
                    f = []
                    for corner in range(8):
                        packed = rv[pl.ds(rb + corner * LANES, LANES)]
                        bf = plsc.bitcast(packed, jnp.bfloat16)
                        f.append(plsc.unpack(
                            bf, format=plsc.PackFormat.INTERLEAVED))
                    for ft in range(F_PER_LEVEL):
                        f03 = f[0][ft] * ox + f[3][ft] * mx
                        f12 = f[1][ft] * ox + f[2][ft] * mx
                        f56 = f[5][ft] * ox + f[6][ft] * mx
                        f47 = f[4][ft] * ox + f[7][ft] * mx
                        f0312 = f03 * oy + f12 * my
                        f4756 = f47 * oy + f56 * my
                        enc = f0312 * oz + f4756 * mz
                        plsc.store_scatter(outv, [p32 + (2 * l + ft)], enc)
                return carry

            lax.fori_loop(np.int32(0), np.int32(GROUPS), interp_group,
                          np.int32(0))
            pltpu.sync_copy(
                outv, out_hbm.at[pl.ds((wbase + c * CHUNK) * NF, CHUNK * NF)])

        A, Bb = bufs

        pass1(np.int32(0), A)
        start_gather(A)

        def pair_body(c2, carry):
            cA = c2 * np.int32(2)
            pass1(cA + 1, Bb)
            start_gather(Bb)
            wait_gather(A)
            pass2(cA, A)
            pass1(cA + 2, A)
            start_gather(A)
            wait_gather(Bb)
            pass2(cA + 1, Bb)
            return carry

        lax.fori_loop(np.int32(0), np.int32(NCHUNK // 2 - 1), pair_body,
                      np.int32(0))

        last = np.int32(NCHUNK - 2)
        pass1(last + 1, Bb)
        start_gather(Bb)
        wait_gather(A)
        pass2(last, A)
        wait_gather(Bb)
        pass2(last + 1, Bb)

    return sc_kernel


_sc_kernel_cache = []


@jax.jit
def _run(in_tensor, hash_table):
    if not _sc_kernel_cache:
        _sc_kernel_cache.append(_make_kernel())
    coords = in_tensor.T
    planes = hash_table.reshape(65536, 128, 2).transpose(0, 2, 1)
    b0 = lax.bitcast_convert_type(
        planes[:, 0, :].astype(jnp.bfloat16), jnp.uint16).astype(jnp.int32)
    b1 = lax.bitcast_convert_type(
        planes[:, 1, :].astype(jnp.bfloat16), jnp.uint16).astype(jnp.int32)
    tpacked = (b0 | (b1 << 16)).reshape(-1)
    flat = _sc_kernel_cache[0](coords[0], coords[1], coords[2], tpacked)
    return flat.reshape(B, NF)


def kernel(in_tensor, hash_table):
    with jax.enable_x64(False):
        return _run(in_tensor, hash_table)


# speedup vs baseline: 16.9954x; 1.0010x over previous
import functools

import jax
import jax.numpy as jnp
import numpy as np
from jax import lax
from jax.experimental import pallas as pl
from jax.experimental.pallas import tpu as pltpu
from jax.experimental.pallas import tpu_sc as plsc

N_LEVELS = 16
F_PER_LEVEL = 2
TABLE_SIZE = 2 ** 19
_GROWTH = np.exp((np.log(4096.0) - np.log(16.0)) / (N_LEVELS - 1))
_SCALINGS = np.floor(16.0 * _GROWTH ** np.arange(N_LEVELS)).astype(np.float32)

_P1 = np.int32(2654435761 - 2 ** 32)
_P2 = np.int32(805459861)
_MASK = np.int32(TABLE_SIZE - 1)

NC = 2
NS = 16
NW = NC * NS
LANES = 16

B = 131072
NF = N_LEVELS * F_PER_LEVEL
CHUNK = 128
GROUPS = CHUNK // LANES
PER_W = B // NW
NCHUNK = PER_W // CHUNK
NIDX = CHUNK * N_LEVELS * 8
ROWSTRIDE = N_LEVELS * 128


def _make_kernel():
    mesh = plsc.VectorSubcoreMesh(core_axis_name="c", subcore_axis_name="s")

    @functools.partial(
        pl.kernel,
        mesh=mesh,
        compiler_params=pltpu.CompilerParams(needs_layout_passes=False),
        out_type=jax.ShapeDtypeStruct((B * NF,), jnp.float32),
        scratch_types=[
            pltpu.VMEM((PER_W,), jnp.float32),
            pltpu.VMEM((PER_W,), jnp.float32),
            pltpu.VMEM((PER_W,), jnp.float32),
            pltpu.VMEM((N_LEVELS * CHUNK,), jnp.float32),
            pltpu.VMEM((N_LEVELS * CHUNK,), jnp.float32),
            pltpu.VMEM((N_LEVELS * CHUNK,), jnp.float32),
            pltpu.VMEM((N_LEVELS * CHUNK,), jnp.float32),
            pltpu.VMEM((N_LEVELS * CHUNK,), jnp.float32),
            pltpu.VMEM((N_LEVELS * CHUNK,), jnp.float32),
            pltpu.VMEM((NIDX,), jnp.int32),
            pltpu.VMEM((NIDX,), jnp.int32),
            pltpu.VMEM((NIDX,), jnp.int32),
            pltpu.VMEM((NIDX,), jnp.int32),
            pltpu.VMEM((CHUNK * NF,), jnp.float32),
            pltpu.VMEM((CHUNK * NF,), jnp.float32),
            pltpu.SemaphoreType.DMA,
            pltpu.SemaphoreType.DMA,
        ],
    )
    def sc_kernel(xs_hbm, ys_hbm, zs_hbm, table_hbm, out_hbm,
                  xv, yv, zv,
                  oxA, oyA, ozA, oxB, oyB, ozB,
                  idxA, idxB, rowsA, rowsB,
                  outA, outB, semA, semB):
        wid = lax.axis_index("s") * NC + lax.axis_index("c")
        iota = lax.iota(jnp.int32, LANES)
        wbase = wid * PER_W

        pltpu.sync_copy(xs_hbm.at[pl.ds(wbase, PER_W)], xv)
        pltpu.sync_copy(ys_hbm.at[pl.ds(wbase, PER_W)], yv)
        pltpu.sync_copy(zs_hbm.at[pl.ds(wbase, PER_W)], zv)

        bufs = (
            (oxA, oyA, ozA, idxA, rowsA, outA, semA),
            (oxB, oyB, ozB, idxB, rowsB, outB, semB),
        )

        def pass1(c, buf):
            ox_, oy_, oz_, i0 = buf[0], buf[1], buf[2], buf[3]
            cbase = c * CHUNK

            def hash_group(g, carry):
                s = g * LANES
                x16 = xv[pl.ds(cbase + s, LANES)]
                y16 = yv[pl.ds(cbase + s, LANES)]
                z16 = zv[pl.ds(cbase + s, LANES)]
                for l in range(N_LEVELS):
                    sl = float(_SCALINGS[l])
                    sx = x16 * sl
                    sy = y16 * sl
                    sz = z16 * sl
                    xf = sx.astype(jnp.int32)
                    yf = sy.astype(jnp.int32)
                    zf = sz.astype(jnp.int32)
                    xff = xf.astype(jnp.float32)
                    yff = yf.astype(jnp.float32)
                    zff = zf.astype(jnp.float32)
                    ox_[pl.ds(l * CHUNK + s, LANES)] = sx - xff
                    oy_[pl.ds(l * CHUNK + s, LANES)] = sy - yff
                    oz_[pl.ds(l * CHUNK + s, LANES)] = sz - zff
                    xc = jnp.where(sx > xff, xf + 1, xf)
                    yc = jnp.where(sy > yff, yf + 1, yf)
                    zc = jnp.where(sz > zff, zf + 1, zf)
                    hyf = yf * _P1
                    hyc = yc * _P1
                    hzf = zf * _P2
                    hzc = zc * _P2
                    ycc = hyc ^ hzc
                    yfc = hyf ^ hzc
                    yff_ = hyf ^ hzf
                    ycf = hyc ^ hzf
                    lofs = np.int32(l * TABLE_SIZE)
                    rb = g * ROWSTRIDE + l * 128
                    combos = (xc ^ ycc, xc ^ yfc, xf ^ yfc, xf ^ ycc,
                              xc ^ ycf, xc ^ yff_, xf ^ yff_, xf ^ ycf)
                    for corner, hv in enumerate(combos):
                        i0[pl.ds(rb + corner * LANES, LANES)] = (
                            (hv & _MASK) + lofs)
                return carry

            lax.fori_loop(np.int32(0), np.int32(GROUPS), hash_group,
                          np.int32(0))

        def start_gather(buf):
            i0, rv, sem = buf[3], buf[4], buf[6]
            pltpu.async_copy(table_hbm.at[i0], rv, sem)

        def wait_gather(buf):
            i0, rv, sem = buf[3], buf[4], buf[6]
            pltpu.make_async_copy(table_hbm.at[i0], rv, sem).wait()

        def pass2(c, buf):
            ox_, oy_, oz_, rv, outv = (buf[0], buf[1], buf[2],
                                       buf[4], buf[5])

            def interp_group(g, carry):
                s = g * LANES
                p32 = (iota + s) * NF
                for l in range(N_LEVELS):
                    ox = ox_[pl.ds(l * CHUNK + s, LANES)]
                    oy = oy_[pl.ds(l * CHUNK + s, LANES)]
                    oz = oz_[pl.ds(l * CHUNK + s, LANES)]
                    mx = 1.0 - ox
                    my = 1.0 - oy
                    mz = 1.0 - oz
                    rb = g * ROWSTRIDE + l * 128
                    f = []
                    for corner in range(8):
                        packed = rv[pl.ds(rb + corner * LANES, LANES)]
                        bf = plsc.bitcast(packed, jnp.bfloat16)
                        f.append(plsc.unpack(
                            bf, format=plsc.PackFormat.INTERLEAVED))
                    for ft in range(F_PER_LEVEL):
                        f03 = f[0][ft] * ox + f[3][ft] * mx
                        f12 = f[1][ft] * ox + f[2][ft] * mx
                        f56 = f[5][ft] * ox + f[6][ft] * mx
                        f47 = f[4][ft] * ox + f[7][ft] * mx
                        f0312 = f03 * oy + f12 * my
                        f4756 = f47 * oy + f56 * my
                        enc = f0312 * oz + f4756 * mz
                        plsc.store_scatter(outv, [p32 + (2 * l + ft)], enc)
                return carry

            lax.fori_loop(np.int32(0), np.int32(GROUPS), interp_group,
                          np.int32(0))
            pltpu.sync_copy(
                outv, out_hbm.at[pl.ds((wbase + c * CHUNK) * NF, CHUNK * NF)])

        A, Bb = bufs

        pass1(np.int32(0), A)
        start_gather(A)

        def pair_body(c2, carry):
            cA = c2 * np.int32(2)
            pass1(cA + 1, Bb)
            start_gather(Bb)
            wait_gather(A)
            pass2(cA, A)
            pass1(cA + 2, A)
            start_gather(A)
            wait_gather(Bb)
            pass2(cA + 1, Bb)
            return carry

        lax.fori_loop(np.int32(0), np.int32(NCHUNK // 2 - 1), pair_body,
                      np.int32(0))

        last = np.int32(NCHUNK - 2)
        pass1(last + 1, Bb)
        start_gather(Bb)
        wait_gather(A)
        pass2(last, A)
        wait_gather(Bb)
        pass2(last + 1, Bb)

    return sc_kernel


_sc_kernel_cache = []


@jax.jit
def _run(in_tensor, hash_table):
    if not _sc_kernel_cache:
        _sc_kernel_cache.append(_make_kernel())
    coords = in_tensor.T
    planes = hash_table.reshape(65536, 128, 2).transpose(0, 2, 1)
    b0 = lax.bitcast_convert_type(
        planes[:, 0, :].astype(jnp.bfloat16), jnp.uint16).astype(jnp.int32)
    b1 = lax.bitcast_convert_type(
        planes[:, 1, :].astype(jnp.bfloat16), jnp.uint16).astype(jnp.int32)
    tpacked = (b0 | (b1 << 16)).reshape(-1)
    flat = _sc_kernel_cache[0](coords[0], coords[1], coords[2], tpacked)
    return flat.reshape(B, NF)


def kernel(in_tensor, hash_table):
    with jax.enable_x64(False):
        return _run(in_tensor, hash_table)
